# Initial kernel scaffold; baseline (speedup 1.0000x reference)
import jax
import jax.numpy as jnp
from jax.experimental import pallas as pl


def kernel(x, tables, linear_w, bias):
    raise NotImplementedError("write your pallas kernel here")


# speedup vs baseline: 36.5450x; 36.5450x over previous
import functools

import jax
import jax.numpy as jnp
from jax import lax
from jax.experimental import pallas as pl
from jax.experimental.pallas import tpu as pltpu
from jax.experimental.pallas import tpu_sc as plsc

F = 26
CARD = 1000
D = 16
B = 4096

NC = 2
NS = 16
NW = NC * NS
BPW = B // NW
CH = 4
NCH = BPW // CH
ROW = F * D


def _lanesum(v):
    lanes = lax.iota(jnp.int32, 16)
    for sh in (8, 4, 2, 1):
        v = v + jnp.take(v, lanes ^ sh, mode="fill")
    return v


def _ffm_body(tab, xf, xft, lw, out, idx_v, rows_v, xft_v, lw_v, out_v,
              sem0, sem1):
    wid = lax.axis_index("s") * NC + lax.axis_index("c")
    base = wid * BPW

    pltpu.sync_copy(lw, lw_v)
    pltpu.sync_copy(xft.at[:, pl.ds(base, BPW)], xft_v)

    sems = (sem0, sem1)

    def start(t, k):
        off = (base + t * CH) * F
        pltpu.sync_copy(xf.at[pl.ds(off, CH * F)], idx_v.at[k])
        pltpu.async_copy(tab.at[idx_v.at[k]], rows_v.at[k], sems[k])

    def wait(k):
        pltpu.make_async_copy(tab.at[idx_v.at[k]], rows_v.at[k],
                              sems[k]).wait()

    start(0, 0)
    start(1, 1)

    def chunk(t, k):
        wait(k)

        def per_b(c, _):
            rb = c * F
            acc = jnp.zeros((16,), jnp.float32)
            for i in range(F):
                for j in range(i + 1, F):
                    a = rows_v[k, rb + i, pl.ds(j * D, D)]
                    b = rows_v[k, rb + j, pl.ds(i * D, D)]
                    acc = acc + a * b
            pos = t * CH + c
            lane = lax.rem(pos, 16)
            grp = pos - lane
            cur = out_v[pl.ds(grp, 16)]
            lanes = lax.iota(jnp.int32, 16)
            out_v[pl.ds(grp, 16)] = jnp.where(lanes == lane, _lanesum(acc), cur)
            return _

        lax.fori_loop(0, CH, per_b, 0)

        @pl.when(t + 2 < NCH)
        def _():
            start(t + 2, k)

    def outer(q, _):
        chunk(2 * q, 0)
        chunk(2 * q + 1, 1)
        return _

    lax.fori_loop(0, NCH // 2, outer, 0)

    for g in range(BPW // 16):
        lin = out_v[pl.ds(g * 16, 16)]
        for i in range(F):
            ids = xft_v[i, pl.ds(g * 16, 16)]
            lin = lin + plsc.load_gather(lw_v, [ids])
        out_v[pl.ds(g * 16, 16)] = lin

    pltpu.sync_copy(out_v, out.at[pl.ds(base, BPW)])


@jax.jit
def _ffm(tab, xf, xft, lw):
    mesh = plsc.VectorSubcoreMesh(core_axis_name="c", subcore_axis_name="s",
                                  num_cores=NC, num_subcores=NS)
    run = pl.kernel(
        _ffm_body,
        out_type=jax.ShapeDtypeStruct((B,), jnp.float32),
        mesh=mesh,
        scratch_types=[
            pltpu.VMEM((2, CH * F), jnp.int32),
            pltpu.VMEM((2, CH * F, ROW), jnp.float32),
            pltpu.VMEM((F, BPW), jnp.int32),
            pltpu.VMEM((F * CARD,), jnp.float32),
            pltpu.VMEM((BPW,), jnp.float32),
            pltpu.SemaphoreType.DMA,
            pltpu.SemaphoreType.DMA,
        ],
        compiler_params=pltpu.CompilerParams(needs_layout_passes=False,
                                             use_tc_tiling_on_sc=False),
    )
    return run(tab, xf, xft, lw)


def kernel(x, tables, linear_w, bias):
    tab = tables.reshape(F * CARD, ROW)
    xf2d = x.astype(jnp.int32) + (jnp.arange(F, dtype=jnp.int32) * CARD)[None, :]
    xf = xf2d.reshape(-1)
    xft = xf2d.T
    lw = linear_w.reshape(-1)
    out = _ffm(tab, xf, xft, lw)
    return out[:, None] + bias


# speedup vs baseline: 93.7011x; 2.5640x over previous
import numpy as np

import jax
import jax.numpy as jnp
from jax import lax
from jax.experimental import pallas as pl
from jax.experimental.pallas import tpu as pltpu
from jax.experimental.pallas import tpu_sc as plsc

F = 26
CARD = 1000
D = 16
B = 4096

NC = 2
NS = 16
NW = NC * NS
BPW = B // NW
NG = B // 16

_PAIRS = [(i, j) for i in range(F) for j in range(i + 1, F)]
NPAIR = len(_PAIRS)
PPT = -(-NPAIR // NW)
_PACKED = np.full((((NPAIR + 7) // 8) * 8,), 0, np.int32)
for _p, (_i, _j) in enumerate(_PAIRS):
    _PACKED[_p] = (_i << 5) | _j
NPAD = _PACKED.shape[0]


def _ffm_body(tt, xft, lw, out, blk_v, xcol_v, acc_v, xft_v, lw_v,
              rows_idx, shared, pairs_s, sem0, sem1):
    cid = lax.axis_index("c")
    sid = lax.axis_index("s")
    wid = sid * NC + cid
    base = wid * BPW

    for p in range(NPAD):
        pairs_s[p] = jnp.int32(_PACKED[p])
    pltpu.sync_copy(lw, lw_v)
    pltpu.sync_copy(xft.at[:, pl.ds(base, BPW)], xft_v)

    sems = (sem0, sem1)

    def pair_ij(p):
        v = pairs_s[p]
        return lax.shift_right_logical(v, 5), lax.bitwise_and(v, 31)

    def start(q, k):
        p = q * NW + wid

        @pl.when(p < NPAIR)
        def _():
            i, j = pair_ij(p)
            pltpu.sync_copy(xft.at[i], xcol_v.at[k, 0])
            pltpu.sync_copy(xft.at[j], xcol_v.at[k, 1])
            pltpu.async_copy(tt.at[i, j], blk_v.at[k, 0], sems[k])
            pltpu.async_copy(tt.at[j, i], blk_v.at[k, 1], sems[k])

    def wait(q, k):
        p = q * NW + wid

        @pl.when(p < NPAIR)
        def _():
            i, j = pair_ij(p)
            pltpu.make_async_copy(tt.at[i, j], blk_v.at[k, 0], sems[k]).wait()
            pltpu.make_async_copy(tt.at[j, i], blk_v.at[k, 1], sems[k]).wait()

    zeros = jnp.zeros((16,), jnp.float32)
    lanes16 = lax.iota(jnp.int32, 16)

    def zero_g(g, _):
        acc_v[g] = zeros
        return _

    lax.fori_loop(0, NG, zero_g, 0)
    for g in range(NG // 16):
        rows_idx[pl.ds(g * 16, 16)] = lanes16 + (g * 16)

    start(0, 0)
    start(1, 1)

    dvecs = [jnp.full((16,), d, jnp.int32) for d in range(D)]

    def do_pair(q, k):
        wait(q, k)
        p = q * NW + wid

        @pl.when(p < NPAIR)
        def _():
            i, j = pair_ij(p)
            ioff = i * CARD
            joff = j * CARD

            def per_group(g, _):
                ci = xcol_v[k, 0, pl.ds(g * 16, 16)] - ioff
                cj = xcol_v[k, 1, pl.ds(g * 16, 16)] - joff
                acc = acc_v[g]
                for d in range(D):
                    u = plsc.load_gather(blk_v.at[k, 0], [dvecs[d], ci])
                    v = plsc.load_gather(blk_v.at[k, 1], [dvecs[d], cj])
                    acc = acc + u * v
                acc_v[g] = acc
                return _

            lax.fori_loop(0, NG, per_group, 0)

        start(q + 2, k)

    def outer(qq, _):
        do_pair(2 * qq, 0)
        do_pair(2 * qq + 1, 1)
        return _

    lax.fori_loop(0, (PPT + 1) // 2, outer, 0)

    grow = base // 16
    for g in range(BPW // 16):
        lin = acc_v[grow + g]
        for i in range(F):
            ids = xft_v[i, pl.ds(g * 16, 16)]
            lin = lin + plsc.load_gather(lw_v, [ids])
        acc_v[grow + g] = lin

    plsc.subcore_barrier()

    @pl.when(sid == 0)
    def _():
        pltpu.sync_copy(acc_v, shared)

    plsc.subcore_barrier()

    @pl.when(sid != 0)
    def _():
        pltpu.sync_copy(acc_v, shared.at[rows_idx], add=True)

    plsc.subcore_barrier()

    @pl.when(sid == 0)
    def _():
        pltpu.sync_copy(shared, out.at[cid])


@jax.jit
def _ffm(tt, xft, lw):
    mesh = plsc.VectorSubcoreMesh(core_axis_name="c", subcore_axis_name="s",
                                  num_cores=NC, num_subcores=NS)
    run = pl.kernel(
        _ffm_body,
        out_type=jax.ShapeDtypeStruct((NC, NG, 16), jnp.float32),
        mesh=mesh,
        scratch_types=[
            pltpu.VMEM((2, 2, D, CARD), jnp.float32),
            pltpu.VMEM((2, 2, B), jnp.int32),
            pltpu.VMEM((NG, 16), jnp.float32),
            pltpu.VMEM((F, BPW), jnp.int32),
            pltpu.VMEM((F * CARD,), jnp.float32),
            pltpu.VMEM((NG,), jnp.int32),
            pltpu.VMEM_SHARED((NG, 16), jnp.float32),
            pltpu.SMEM((NPAD,), jnp.int32),
            pltpu.SemaphoreType.DMA,
            pltpu.SemaphoreType.DMA,
        ],
        compiler_params=pltpu.CompilerParams(needs_layout_passes=False,
                                             use_tc_tiling_on_sc=False),
    )
    return run(tt, xft, lw)


def kernel(x, tables, linear_w, bias):
    tt = jnp.transpose(tables, (0, 2, 3, 1))
    xft = x.astype(jnp.int32).T + (jnp.arange(F, dtype=jnp.int32) * CARD)[:, None]
    lw = linear_w.reshape(-1)
    part = _ffm(tt, xft, lw)
    out = (part[0] + part[1]).reshape(B)
    return out[:, None] + bias
